# Initial kernel scaffold; baseline (speedup 1.0000x reference)
import jax
import jax.numpy as jnp
from jax.experimental import pallas as pl


def kernel(feat_src, feat_dst, metapath_idx, attn_r):
    raise NotImplementedError("write your pallas kernel here")


# speedup vs baseline: 39.7490x; 39.7490x over previous
import functools

import jax
import jax.numpy as jnp
from jax import lax
from jax.experimental import pallas as pl
from jax.experimental.pallas import tpu as pltpu
from jax.experimental.pallas import tpu_sc as plsc

_NC = 2
_NS = 16
_C = 80


def _w_body(x_ref, wmat_ref, o_ref):
    er = jnp.dot(x_ref[...], wmat_ref[...], preferred_element_type=jnp.float32)
    e = jnp.where(er > 0, er, 0.01 * er)
    o_ref[...] = jnp.exp(e)


def kernel(feat_src, feat_dst, metapath_idx, attn_r):
    M, HD = feat_src.shape
    N = feat_dst.shape[0]
    H, D = attn_r.shape[1], attn_r.shape[2]
    HP = 16

    seg = metapath_idx[:, 0]

    r_flat = attn_r.reshape(H * D).astype(jnp.float32)
    j = jnp.arange(HD)
    wmat = jnp.zeros((HD, HP), jnp.float32).at[j, j // D].set(r_flat)

    BM = 1280
    w_edges = pl.pallas_call(
        _w_body,
        grid=(M // BM,),
        in_specs=[
            pl.BlockSpec((BM, HD), lambda i: (i, 0)),
            pl.BlockSpec((HD, HP), lambda i: (0, 0)),
        ],
        out_specs=pl.BlockSpec((BM, HP), lambda i: (i, 0)),
        out_shape=jax.ShapeDtypeStruct((M, HP), jnp.float32),
    )(feat_src, wmat)

    n_work = _NC * _NS
    m_per = M // n_work
    nchunk = m_per // _C
    nz = N // _C
    DR = N * H // HD

    mesh = plsc.VectorSubcoreMesh(core_axis_name="c", subcore_axis_name="s")

    @functools.partial(
        pl.kernel,
        out_type=jax.ShapeDtypeStruct((_NC, N, HD), jnp.float32),
        mesh=mesh,
        compiler_params=pltpu.CompilerParams(needs_layout_passes=False),
        scratch_types=[
            pltpu.VMEM((_C, HD), jnp.float32),
            pltpu.VMEM((_C, HP), jnp.float32),
            pltpu.VMEM((_C, HD), jnp.float32),
            pltpu.VMEM((_C,), jnp.int32),
            pltpu.VMEM_SHARED((N, HD), jnp.float32),
        ],
    )
    def sc_feats(x_hbm, w_hbm, seg_hbm, outf_hbm, xbuf, wbuf, ybuf, idxbuf,
                 acc):
        c = lax.axis_index("c")
        s = lax.axis_index("s")
        wid = s * _NC + c
        zero = jnp.zeros((HP,), jnp.float32)

        @pl.loop(0, _C)
        def _(r):
            for k in range(HD // HP):
                ybuf[r, k * HP:(k + 1) * HP] = zero

        @pl.loop(s, nz, step=_NS)
        def _(q):
            off = pl.multiple_of(q * _C, 8)
            pltpu.sync_copy(ybuf, acc.at[pl.ds(off, _C)])
        plsc.subcore_barrier()

        base0 = wid * m_per

        @pl.loop(0, nchunk)
        def _(i):
            base = pl.multiple_of(base0 + i * _C, 8)
            pltpu.sync_copy(x_hbm.at[pl.ds(base, _C)], xbuf)
            pltpu.sync_copy(w_hbm.at[pl.ds(base, _C)], wbuf)
            pltpu.sync_copy(seg_hbm.at[pl.ds(base, _C)], idxbuf)

            @pl.loop(0, _C)
            def _(e):
                wv = wbuf[e, :]
                for h in range(H):
                    ybuf[e, h * D:(h + 1) * D] = (
                        xbuf[e, h * D:(h + 1) * D] * wv[h])

            pltpu.sync_copy(ybuf, acc.at[idxbuf], add=True)

        plsc.subcore_barrier()

        @pl.loop(s, nz, step=_NS)
        def _(q):
            off = pl.multiple_of(q * _C, 8)
            pltpu.sync_copy(acc.at[pl.ds(off, _C)], outf_hbm.at[c, pl.ds(off, _C)])

    parts_f = sc_feats(feat_src, w_edges, seg)

    @functools.partial(
        pl.kernel,
        out_type=jax.ShapeDtypeStruct((n_work, N * H), jnp.float32),
        mesh=mesh,
        compiler_params=pltpu.CompilerParams(needs_layout_passes=False),
        scratch_types=[
            pltpu.VMEM((_C, HP), jnp.float32),
            pltpu.VMEM((_C,), jnp.int32),
            pltpu.VMEM((N * H,), jnp.float32),
        ],
    )
    def sc_denom(w_hbm, seg_hbm, outd_hbm, wbuf, idxbuf, dbuf):
        c = lax.axis_index("c")
        s = lax.axis_index("s")
        wid = s * _NC + c
        zero = jnp.zeros((HP,), jnp.float32)
        iota = lax.iota(jnp.int32, HP)
        hmask = iota < H

        @pl.loop(0, N * H // HP)
        def _(r):
            dbuf[pl.ds(r * HP, HP)] = zero

        base0 = wid * m_per

        @pl.loop(0, nchunk)
        def _(i):
            base = pl.multiple_of(base0 + i * _C, 8)
            pltpu.sync_copy(w_hbm.at[pl.ds(base, _C)], wbuf)
            pltpu.sync_copy(seg_hbm.at[pl.ds(base, _C)], idxbuf)

            @pl.loop(0, _C // HP)
            def _(g):
                iv = idxbuf[pl.ds(g * HP, HP)]
                for e16 in range(HP):
                    wv = wbuf[g * HP + e16, :]
                    flat = iv[e16] * H + iota
                    plsc.addupdate_scatter(dbuf, [flat], wv, mask=hmask)

        pltpu.sync_copy(dbuf, outd_hbm.at[wid])

    parts_d = sc_denom(w_edges, seg)

    NPR = HD // H
    FW = NPR * HD
    NR = N // NPR
    cols = jnp.arange(FW)
    expand = (jnp.arange(HD)[:, None]
              == (cols // HD) * H + (cols % HD) // D).astype(jnp.float32)

    p0v = parts_f[0].reshape(NR, FW)
    p1v = parts_f[1].reshape(NR, FW)
    den_parts = parts_d.reshape(n_work, DR, HD)

    def _combine_body(p0_ref, p1_ref, dp_ref, ex_ref, o_ref):
        u = p0_ref[...] + p1_ref[...]
        den = jnp.sum(dp_ref[...], axis=0)
        den_rep = jnp.dot(den, ex_ref[...],
                          preferred_element_type=jnp.float32)
        v = u / jnp.where(den_rep > 0, den_rep, 1.0)
        o_ref[...] = jnp.where(v > 0, v, jnp.exp(v) - 1.0)

    out = pl.pallas_call(
        _combine_body,
        grid=(1,),
        in_specs=[
            pl.BlockSpec((NR, FW), lambda i: (0, 0)),
            pl.BlockSpec((NR, FW), lambda i: (0, 0)),
            pl.BlockSpec((n_work, NR, HD), lambda i: (0, 0, 0)),
            pl.BlockSpec((HD, FW), lambda i: (0, 0)),
        ],
        out_specs=pl.BlockSpec((NR, FW), lambda i: (0, 0)),
        out_shape=jax.ShapeDtypeStruct((NR, FW), jnp.float32),
    )(p0v, p1v, den_parts, expand)
    return out.reshape(N, HD)
